# baseline (speedup 1.0000x reference)
import functools

import jax
import jax.numpy as jnp
from jax import lax
from jax.experimental import pallas as pl
from jax.experimental.pallas import tpu as pltpu
from jax.experimental.pallas import tpu_sc as plsc

VOCAB = 1000000
EMBED_DIM = 64
BATCH = 4096
HIST = 200
B = BATCH * HIST

_INFO = plsc.get_sparse_core_info()
_NC = _INFO.num_cores
_NS = _INFO.num_subcores
_NW = _NC * _NS
_BPW = B // _NW
_BATW = BATCH // _NW
_C = HIST
_NCHUNK = _BATW
_NBUF = 4
_LA = 2
_NROUND = _NCHUNK // _NBUF


@functools.partial(
    pl.kernel,
    mesh=plsc.VectorSubcoreMesh(core_axis_name="c", subcore_axis_name="s"),
    out_type=jax.ShapeDtypeStruct((BATCH, HIST, EMBED_DIM), jnp.float32),
    scratch_types=[
        pltpu.VMEM((_BPW,), jnp.int32),
        pltpu.VMEM((_NBUF, _C, EMBED_DIM), jnp.float32),
        [pltpu.SemaphoreType.DMA] * _NBUF,
        [pltpu.SemaphoreType.DMA] * _NBUF,
    ],
    compiler_params=pltpu.CompilerParams(use_tc_tiling_on_sc=False),
)
def _gather_kernel(words_hbm, table_hbm, out_hbm, idx_v, rows_v, gsems, wsems):
    wid = lax.axis_index("s") * _NC + lax.axis_index("c")
    base = wid * _BPW
    bat0 = wid * _BATW

    pltpu.sync_copy(words_hbm.at[pl.ds(base, _BPW)], idx_v)

    def start_gather(k, b):
        pltpu.make_async_copy(
            table_hbm.at[idx_v.at[pl.ds(k * _C, _C)]], rows_v.at[b], gsems[b]
        ).start()

    def wait_gather(b):
        pltpu.make_async_copy(
            table_hbm.at[idx_v.at[pl.ds(0, _C)]], rows_v.at[b], gsems[b]
        ).wait()

    def start_write(k, b):
        pltpu.make_async_copy(
            rows_v.at[b], out_hbm.at[bat0 + k], wsems[b]
        ).start()

    def wait_write(b):
        pltpu.make_async_copy(
            rows_v.at[b], out_hbm.at[bat0], wsems[b]
        ).wait()

    for k in range(_LA):
        start_gather(k, k % _NBUF)

    for b in range(_NBUF):
        k = b
        wait_gather(b)
        start_write(k, b)
        j = k + _LA
        jb = j % _NBUF
        if j - _NBUF >= 0:
            wait_write(jb)
        start_gather(j, jb)

    def round_body(r, carry):
        k0 = r * _NBUF
        for b in range(_NBUF):
            k = k0 + b
            wait_gather(b)
            start_write(k, b)
            j = k + _LA
            jb = (b + _LA) % _NBUF
            wait_write(jb)
            start_gather(j, jb)
        return carry

    lax.fori_loop(1, _NROUND - 1, round_body, 0)

    k0 = (_NROUND - 1) * _NBUF
    for b in range(_NBUF):
        k = k0 + b
        wait_gather(b)
        start_write(k, b)
        j = k + _LA
        if j < _NCHUNK:
            jb = (b + _LA) % _NBUF
            wait_write(jb)
            start_gather(j, jb)

    for b in range(_NBUF):
        wait_write(b)


def kernel(words, table):
    return _gather_kernel(words.reshape(B), table)


# speedup vs baseline: 1.0737x; 1.0737x over previous
import functools

import jax
import jax.numpy as jnp
from jax import lax
from jax.experimental import pallas as pl
from jax.experimental.pallas import tpu as pltpu
from jax.experimental.pallas import tpu_sc as plsc

VOCAB = 1000000
EMBED_DIM = 64
BATCH = 4096
HIST = 200
B = BATCH * HIST

_INFO = plsc.get_sparse_core_info()
_NC = _INFO.num_cores
_NS = _INFO.num_subcores
_NW = _NC * _NS
_BPW = B // _NW
_C = 200
_NCHUNK = _BPW // _C
_NBUF = 4
_LA = 2
_NROUND = _NCHUNK // _NBUF

_PAD_BLK = 8000


def _pad_body(t_ref, o_ref):
    x = t_ref[...]
    o_ref[...] = jnp.concatenate([x, x], axis=1)


_pad_table = pl.pallas_call(
    _pad_body,
    grid=(VOCAB // _PAD_BLK,),
    in_specs=[pl.BlockSpec((_PAD_BLK, EMBED_DIM), lambda i: (i, 0))],
    out_specs=pl.BlockSpec((_PAD_BLK, 128), lambda i: (i, 0)),
    out_shape=jax.ShapeDtypeStruct((VOCAB, 128), jnp.float32),
)


@functools.partial(
    pl.kernel,
    mesh=plsc.VectorSubcoreMesh(core_axis_name="c", subcore_axis_name="s"),
    out_type=jax.ShapeDtypeStruct((B, 128), jnp.float32),
    scratch_types=[
        pltpu.VMEM((_BPW,), jnp.int32),
        pltpu.VMEM((_NBUF, _C, 128), jnp.float32),
        [pltpu.SemaphoreType.DMA] * _NBUF,
        [pltpu.SemaphoreType.DMA] * _NBUF,
    ],
    compiler_params=pltpu.CompilerParams(use_tc_tiling_on_sc=False),
)
def _gather_kernel(words_hbm, table_hbm, out_hbm, idx_v, rows_v, gsems, wsems):
    wid = lax.axis_index("s") * _NC + lax.axis_index("c")
    base = wid * _BPW

    pltpu.sync_copy(words_hbm.at[pl.ds(base, _BPW)], idx_v)

    def start_gather(k, b):
        pltpu.make_async_copy(
            table_hbm.at[idx_v.at[pl.ds(k * _C, _C)]], rows_v.at[b], gsems[b]
        ).start()

    def wait_gather(b):
        pltpu.make_async_copy(
            table_hbm.at[idx_v.at[pl.ds(0, _C)]], rows_v.at[b], gsems[b]
        ).wait()

    def start_write(k, b):
        pltpu.make_async_copy(
            rows_v.at[b], out_hbm.at[pl.ds(base + k * _C, _C)], wsems[b]
        ).start()

    def wait_write(b):
        pltpu.make_async_copy(
            rows_v.at[b], out_hbm.at[pl.ds(base, _C)], wsems[b]
        ).wait()

    for k in range(_LA):
        start_gather(k, k % _NBUF)

    for b in range(_NBUF):
        k = b
        wait_gather(b)
        start_write(k, b)
        j = k + _LA
        jb = j % _NBUF
        if j - _NBUF >= 0:
            wait_write(jb)
        start_gather(j, jb)

    def round_body(r, carry):
        k0 = r * _NBUF
        for b in range(_NBUF):
            k = k0 + b
            wait_gather(b)
            start_write(k, b)
            j = k + _LA
            jb = (b + _LA) % _NBUF
            wait_write(jb)
            start_gather(j, jb)
        return carry

    lax.fori_loop(1, _NROUND - 1, round_body, 0)

    k0 = (_NROUND - 1) * _NBUF
    for b in range(_NBUF):
        k = k0 + b
        wait_gather(b)
        start_write(k, b)
        j = k + _LA
        if j < _NCHUNK:
            jb = (b + _LA) % _NBUF
            wait_write(jb)
            start_gather(j, jb)

    for b in range(_NBUF):
        wait_write(b)


def kernel(words, table):
    padtable = _pad_table(table)
    out128 = _gather_kernel(words.reshape(B), padtable)
    return out128[:, :64].reshape(BATCH, HIST, EMBED_DIM)
